# baseline (speedup 1.0000x reference)
import functools

import jax
import jax.numpy as jnp
from jax import lax
from jax.experimental import pallas as pl
from jax.experimental.pallas import tpu as pltpu
from jax.experimental.pallas import tpu_sc as plsc

V = 1000000
D = 64
B = 16384
L = 50

_info = plsc.get_sparse_core_info()
NC = _info.num_cores
NS = _info.num_subcores
NW = NC * NS
BPW = B // NW
NSLOT = 4

_mesh = plsc.VectorSubcoreMesh(core_axis_name="c", subcore_axis_name="s")


@functools.partial(
    pl.kernel,
    mesh=_mesh,
    compiler_params=pltpu.CompilerParams(use_tc_tiling_on_sc=False),
    out_type=jax.ShapeDtypeStruct((B, L, D), jnp.bfloat16),
    scratch_types=[
        pltpu.VMEM((BPW, L), jnp.int32),
        pltpu.VMEM((NSLOT, L, D), jnp.bfloat16),
        pltpu.SemaphoreType.DMA,
        pltpu.SemaphoreType.DMA,
    ],
)
def _emb_lookup(idx_hbm, table_hbm, out_hbm, idx_v, rows_v, gsem, wsem):
    wid = lax.axis_index("s") * NC + lax.axis_index("c")
    base = wid * BPW
    pltpu.sync_copy(idx_hbm.at[pl.ds(base, BPW)], idx_v)

    def gather(b, slot):
        pltpu.async_copy(table_hbm.at[idx_v.at[b]], rows_v.at[slot], gsem)

    def wait_write():
        pltpu.make_async_copy(rows_v.at[0], out_hbm.at[base], wsem).wait()

    for p in range(NSLOT - 1):
        gather(p, p)

    def body(b, carry):
        slot = lax.rem(b, NSLOT)

        @pl.when(b + NSLOT - 1 < BPW)
        def _():
            sp = lax.rem(b + NSLOT - 1, NSLOT)

            @pl.when(b >= 1)
            def _():
                wait_write()

            gather(b + NSLOT - 1, sp)

        pltpu.make_async_copy(
            table_hbm.at[idx_v.at[b]], rows_v.at[slot], gsem).wait()
        pltpu.async_copy(rows_v.at[slot], out_hbm.at[base + b], wsem)
        return carry

    lax.fori_loop(0, BPW, body, 0)

    for _ in range(NSLOT):
        wait_write()


def kernel(indices, table):
    out16 = _emb_lookup(indices.astype(jnp.int32), table.astype(jnp.bfloat16))
    return out16.astype(jnp.float32)


# speedup vs baseline: 1.3913x; 1.3913x over previous
import functools

import jax
import jax.numpy as jnp
from jax import lax
from jax.experimental import pallas as pl
from jax.experimental.pallas import tpu as pltpu
from jax.experimental.pallas import tpu_sc as plsc

V = 1000000
D = 64
B = 16384
L = 50

_info = plsc.get_sparse_core_info()
NC = _info.num_cores
NS = _info.num_subcores
NW = NC * NS
BPW = B // NW
NSLOT = 8

_mesh = plsc.VectorSubcoreMesh(core_axis_name="c", subcore_axis_name="s")


@functools.partial(
    pl.kernel,
    mesh=_mesh,
    compiler_params=pltpu.CompilerParams(use_tc_tiling_on_sc=False),
    out_type=jax.ShapeDtypeStruct((B, L, D), jnp.float32),
    scratch_types=[
        pltpu.VMEM((BPW, L), jnp.int32),
        pltpu.VMEM((NSLOT, L, D), jnp.float32),
        pltpu.SemaphoreType.DMA,
        pltpu.SemaphoreType.DMA,
    ],
)
def _emb_lookup(idx_hbm, table_hbm, out_hbm, idx_v, rows_v, gsem, wsem):
    wid = lax.axis_index("s") * NC + lax.axis_index("c")
    base = wid * BPW
    pltpu.sync_copy(idx_hbm.at[pl.ds(base, BPW)], idx_v)

    def gather(b, slot):
        pltpu.async_copy(table_hbm.at[idx_v.at[b]], rows_v.at[slot], gsem)

    def wait_write():
        pltpu.make_async_copy(rows_v.at[0], out_hbm.at[base], wsem).wait()

    for p in range(NSLOT - 1):
        gather(p, p)

    def body(b, carry):
        slot = lax.rem(b, NSLOT)

        @pl.when(b + NSLOT - 1 < BPW)
        def _():
            sp = lax.rem(b + NSLOT - 1, NSLOT)

            @pl.when(b >= 1)
            def _():
                wait_write()

            gather(b + NSLOT - 1, sp)

        pltpu.make_async_copy(
            table_hbm.at[idx_v.at[b]], rows_v.at[slot], gsem).wait()
        pltpu.async_copy(rows_v.at[slot], out_hbm.at[base + b], wsem)
        return carry

    lax.fori_loop(0, BPW, body, 0)

    for _ in range(NSLOT):
        wait_write()


def kernel(indices, table):
    return _emb_lookup(indices.astype(jnp.int32), table)
